# baseline (speedup 1.0000x reference)
import functools
import jax
import jax.numpy as jnp
from jax import lax
from jax.experimental import pallas as pl
from jax.experimental.pallas import tpu as pltpu
from jax.experimental.pallas import tpu_sc as plsc

N = 10000
E = 320000
D = 128
Q = 32
NS = 16
NC = 2
NP = 2
EPS = E // NS
CH = 128
NCH = EPS // CH
TAIL = EPS - NCH * CH
NPAD = 10240
RPS = NPAD // NS


NR = 4


def _sc_body(xcat_hbm, colcat_hbm, row_hbm, e_hbm, out1_hbm, out2_hbm,
             col0, col1, col2, col3, row0, row1, row2, row3,
             e0, e1, e2, e3,
             rows0, rows1, rows2, rows3, sca0, sca1, sca2, sca3,
             col_t, row_t, e_t, rows_t, scaled_t,
             zbuf, s1s, s2s,
             sed0, sed1, sed2, sed3, sg0, sg1, sg2, sg3,
             ss0, ss1, ss2, ss3, sem):
    c = lax.axis_index("c")
    s = lax.axis_index("s")
    r0 = s * RPS
    base = s * EPS
    zv = jnp.zeros((16,), jnp.float32)

    colb = (col0, col1, col2, col3)
    rowb = (row0, row1, row2, row3)
    eb = (e0, e1, e2, e3)
    rowsb = (rows0, rows1, rows2, rows3)
    scab = (sca0, sca1, sca2, sca3)
    sed = (sed0, sed1, sed2, sed3)
    sg = (sg0, sg1, sg2, sg3)
    ss = (ss0, ss1, ss2, ss3)

    def zero_body(i, carry):
        for k in range(Q // 16):
            zbuf[i, pl.ds(16 * k, 16)] = zv
        return carry

    def drain1i(semr, ref):
        pltpu.make_async_copy(colcat_hbm.at[pl.ds(0, CH)], ref, semr).wait()

    def drain1f(semr, ref):
        pltpu.make_async_copy(e_hbm.at[pl.ds(0, CH)], ref, semr).wait()

    def drain2(semr, ref):
        pltpu.make_async_copy(xcat_hbm.at[pl.ds(0, CH)], ref, semr).wait()

    def scale(rowsref, scaref, eref, n):
        @plsc.parallel_loop(0, n, 1, unroll=8)
        def scale_body(i):
            ev = plsc.load_gather(eref, [jnp.full((16,), i, jnp.int32)])
            for k in range(Q // 16):
                scaref[i, pl.ds(16 * k, 16)] = (
                    rowsref[i, pl.ds(16 * k, 16)] * ev)

    def process_tail(q, g):
        pltpu.sync_copy(colcat_hbm.at[pl.ds(q * E + g, TAIL)], col_t)
        pltpu.sync_copy(row_hbm.at[pl.ds(g, TAIL)], row_t)
        pltpu.sync_copy(e_hbm.at[pl.ds(g, TAIL)], e_t)
        pltpu.async_copy(xcat_hbm.at[col_t], rows_t, sem).wait()
        scale(rows_t, scaled_t, e_t, TAIL)
        pltpu.sync_copy(rows_t, s1s.at[row_t], add=True)
        pltpu.sync_copy(scaled_t, s2s.at[row_t], add=True)

    for p in range(NP):
        q = NP * p + c

        lax.fori_loop(0, RPS, zero_body, 0)
        pltpu.sync_copy(zbuf, s1s.at[pl.ds(r0, RPS)])
        pltpu.sync_copy(zbuf, s2s.at[pl.ds(r0, RPS)])
        plsc.subcore_barrier()

        def edata(t, r):
            g = base + t * CH
            pltpu.async_copy(colcat_hbm.at[pl.ds(q * E + g, CH)],
                             colb[r], sed[r])
            pltpu.async_copy(row_hbm.at[pl.ds(g, CH)], rowb[r], sed[r])
            pltpu.async_copy(e_hbm.at[pl.ds(g, CH)], eb[r], sed[r])

        def wait_edata(r):
            drain1i(sed[r], colb[r])
            drain1i(sed[r], rowb[r])
            drain1f(sed[r], eb[r])

        def gather(r):
            pltpu.async_copy(xcat_hbm.at[colb[r]], rowsb[r], sg[r])

        def scatter(r):
            pltpu.async_copy(rowsb[r], s1s.at[rowb[r]], ss[r], add=True)
            pltpu.async_copy(scab[r], s2s.at[rowb[r]], ss[r], add=True)

        def wait_scatter(r):
            drain2(ss[r], rowsb[r])
            drain2(ss[r], scab[r])

        edata(0, 0)
        wait_edata(0)
        gather(0)
        edata(1, 1)
        wait_edata(1)
        gather(1)
        drain2(sg[0], rowsb[0])
        scale(rowsb[0], scab[0], eb[0], CH)
        scatter(0)
        edata(2, 2)
        wait_edata(2)
        gather(2)
        drain2(sg[1], rowsb[1])
        scale(rowsb[1], scab[1], eb[1], CH)
        scatter(1)
        edata(3, 3)

        def steady(i, carry):
            for k in range(4):
                t = 2 + 4 * i + k
                r = (2 + k) % 4
                r1 = (3 + k) % 4
                r2 = k % 4
                wait_scatter(r2)
                edata(t + 2, r2)
                wait_edata(r1)
                gather(r1)
                drain2(sg[r], rowsb[r])
                scale(rowsb[r], scab[r], eb[r], CH)
                scatter(r)
            return carry

        lax.fori_loop(0, (NCH - 4) // 4, steady, 0)

        wait_edata(3)
        gather(3)
        drain2(sg[2], rowsb[2])
        scale(rowsb[2], scab[2], eb[2], CH)
        scatter(2)
        drain2(sg[3], rowsb[3])
        scale(rowsb[3], scab[3], eb[3], CH)
        scatter(3)
        for r in range(NR):
            wait_scatter(r)

        process_tail(q, base + NCH * CH)

        plsc.subcore_barrier()
        pltpu.sync_copy(s1s.at[pl.ds(r0, RPS)],
                        out1_hbm.at[pl.ds(q * NPAD + r0, RPS)])
        pltpu.sync_copy(s2s.at[pl.ds(r0, RPS)],
                        out2_hbm.at[pl.ds(q * NPAD + r0, RPS)])


_sc_edge_kernel = functools.partial(
    pl.kernel,
    out_type=(jax.ShapeDtypeStruct((2 * NP * NPAD, Q), jnp.float32),
              jax.ShapeDtypeStruct((2 * NP * NPAD, Q), jnp.float32)),
    mesh=plsc.VectorSubcoreMesh(core_axis_name="c", subcore_axis_name="s"),
    scratch_types=(
        [pltpu.VMEM((CH,), jnp.int32)] * 8
        + [pltpu.VMEM((CH,), jnp.float32)] * 4
        + [pltpu.VMEM((CH, Q), jnp.float32)] * 8
        + [
            pltpu.VMEM((TAIL,), jnp.int32),
            pltpu.VMEM((TAIL,), jnp.int32),
            pltpu.VMEM((TAIL,), jnp.float32),
            pltpu.VMEM((TAIL, Q), jnp.float32),
            pltpu.VMEM((TAIL, Q), jnp.float32),
            pltpu.VMEM((RPS, Q), jnp.float32),
            pltpu.VMEM_SHARED((NPAD, Q), jnp.float32),
            pltpu.VMEM_SHARED((NPAD, Q), jnp.float32),
        ]
        + [pltpu.SemaphoreType.DMA] * 13
    ),
    compiler_params=pltpu.CompilerParams(use_tc_tiling_on_sc=False,
                                         needs_layout_passes=False),
)(_sc_body)


JP = 4


def _tc_body(xT_ref, s1t_ref, s2t_ref, at_ref, bt_ref, wlt_ref, wtt_ref,
             out_ref, aggt_ref):
    t = pl.program_id(0)

    @pl.when(t == 0)
    def _init():
        aggt_ref[...] = (
            jnp.dot(at_ref[...], s2t_ref[...],
                    preferred_element_type=jnp.float32)
            + jnp.dot(bt_ref[...], s1t_ref[...],
                      preferred_element_type=jnp.float32))
        out_ref[...] = xT_ref[...] + jnp.dot(
            wlt_ref[...], xT_ref[...], preferred_element_type=jnp.float32)

    ypair = jnp.dot(wtt_ref[0], xT_ref[...],
                    preferred_element_type=jnp.float32)
    acc = out_ref[...]
    for u in range(JP):
        acc += (ypair[u * D:(u + 1) * D, :]
                * aggt_ref[pl.ds(t * JP + u, 1), :])
    out_ref[...] = acc


def _tc_update(xT, s1t, s2t, AT, BT, WlT, WtT):
    return pl.pallas_call(
        _tc_body,
        grid=(D // JP,),
        in_specs=[
            pl.BlockSpec((D, N), lambda t: (0, 0)),
            pl.BlockSpec((D, N), lambda t: (0, 0)),
            pl.BlockSpec((D, N), lambda t: (0, 0)),
            pl.BlockSpec((D, D), lambda t: (0, 0)),
            pl.BlockSpec((D, D), lambda t: (0, 0)),
            pl.BlockSpec((D, D), lambda t: (0, 0)),
            pl.BlockSpec((1, JP * D, D), lambda t: (t, 0, 0)),
        ],
        out_specs=pl.BlockSpec((D, N), lambda t: (0, 0)),
        out_shape=jax.ShapeDtypeStruct((D, N), jnp.float32),
        scratch_shapes=[pltpu.VMEM((D, N), jnp.float32)],
        compiler_params=pltpu.CompilerParams(
            dimension_semantics=("arbitrary",)),
    )(xT, s1t, s2t, AT, BT, WlT, WtT)


@jax.jit
def kernel(node_features, edge_index, edge_attr_tensor, node_attr_scalar_raw,
           W_tp_msg, W_lin_msg, W_tp_upd, W_lin_upd):
    x = node_features
    col = edge_index[1]
    row = edge_index[0]
    e = edge_attr_tensor[:, 3]

    colcat = jnp.concatenate(
        [col, col + N, col + 2 * N, col + 3 * N])
    xcat = jnp.concatenate(
        [x[:, 0:Q], x[:, Q:2 * Q], x[:, 2 * Q:3 * Q], x[:, 3 * Q:4 * Q]],
        axis=0)

    out1, out2 = _sc_edge_kernel(xcat, colcat, row, e)

    s1t = out1.reshape(4, NPAD, Q)[:, :N].transpose(0, 2, 1).reshape(D, N)
    s2t = out2.reshape(4, NPAD, Q)[:, :N].transpose(0, 2, 1).reshape(D, N)

    xT = x.T
    scale = 1.0 / (jnp.sqrt(float(D)) * float(D))
    AT = W_tp_msg[:, 0, :].T * scale
    BT = W_lin_msg.T * scale
    WlT = W_lin_upd.T / jnp.sqrt(float(D))
    WtT = W_tp_upd.transpose(1, 2, 0).reshape(D // JP, JP * D, D)

    outT = _tc_update(xT, s1t, s2t, AT, BT, WlT, WtT)
    return outT.T


# speedup vs baseline: 1.0186x; 1.0186x over previous
import functools
import jax
import jax.numpy as jnp
from jax import lax
from jax.experimental import pallas as pl
from jax.experimental.pallas import tpu as pltpu
from jax.experimental.pallas import tpu_sc as plsc

N = 10000
E = 320000
D = 128
Q = 32
NS = 16
NC = 2
NP = 2
EPS = E // NS
CH = 128
NCH = EPS // CH
TAIL = EPS - NCH * CH
NPAD = 10240
RPS = NPAD // NS


NR = 4


def _sc_body(xcat_hbm, colcat_hbm, row_hbm, e_hbm, out1_hbm, out2_hbm,
             col0, col1, col2, col3, row0, row1, row2, row3,
             e0, e1, e2, e3,
             rows0, rows1, rows2, rows3, sca0, sca1, sca2, sca3,
             col_t, row_t, e_t, rows_t, scaled_t,
             zbuf, s1s, s2s,
             sed0, sed1, sed2, sed3, sg0, sg1, sg2, sg3,
             ss0, ss1, ss2, ss3, sem):
    c = lax.axis_index("c")
    s = lax.axis_index("s")
    r0 = s * RPS
    base = s * EPS
    zv = jnp.zeros((16,), jnp.float32)

    colb = (col0, col1, col2, col3)
    rowb = (row0, row1, row2, row3)
    eb = (e0, e1, e2, e3)
    rowsb = (rows0, rows1, rows2, rows3)
    scab = (sca0, sca1, sca2, sca3)
    sed = (sed0, sed1, sed2, sed3)
    sg = (sg0, sg1, sg2, sg3)
    ss = (ss0, ss1, ss2, ss3)

    def zero_body(i, carry):
        for k in range(Q // 16):
            zbuf[i, pl.ds(16 * k, 16)] = zv
        return carry

    def drain1i(semr, ref):
        pltpu.make_async_copy(colcat_hbm.at[pl.ds(0, CH)], ref, semr).wait()

    def drain1f(semr, ref):
        pltpu.make_async_copy(e_hbm.at[pl.ds(0, CH)], ref, semr).wait()

    def drain2(semr, ref):
        pltpu.make_async_copy(xcat_hbm.at[pl.ds(0, CH)], ref, semr).wait()

    def scale(rowsref, scaref, eref, n):
        @plsc.parallel_loop(0, n // 16, 1, unroll=1)
        def scale_body(g):
            evec = eref[pl.ds(16 * g, 16)]
            for u in range(16):
                ev = lax.gather(
                    evec, jnp.full((16, 1), u, jnp.int32),
                    lax.GatherDimensionNumbers(
                        offset_dims=(), collapsed_slice_dims=(0,),
                        start_index_map=(0,)),
                    (1,), mode=lax.GatherScatterMode.PROMISE_IN_BOUNDS)
                i = 16 * g + u
                for k in range(Q // 16):
                    scaref[i, pl.ds(16 * k, 16)] = (
                        rowsref[i, pl.ds(16 * k, 16)] * ev)

    def process_tail(q, g):
        pltpu.sync_copy(colcat_hbm.at[pl.ds(q * E + g, TAIL)], col_t)
        pltpu.sync_copy(row_hbm.at[pl.ds(g, TAIL)], row_t)
        pltpu.sync_copy(e_hbm.at[pl.ds(g, TAIL)], e_t)
        pltpu.async_copy(xcat_hbm.at[col_t], rows_t, sem).wait()
        scale(rows_t, scaled_t, e_t, TAIL)
        pltpu.sync_copy(rows_t, s1s.at[row_t], add=True)
        pltpu.sync_copy(scaled_t, s2s.at[row_t], add=True)

    for p in range(NP):
        q = NP * p + c

        lax.fori_loop(0, RPS, zero_body, 0)
        pltpu.sync_copy(zbuf, s1s.at[pl.ds(r0, RPS)])
        pltpu.sync_copy(zbuf, s2s.at[pl.ds(r0, RPS)])
        plsc.subcore_barrier()

        def edata(t, r):
            g = base + t * CH
            pltpu.async_copy(colcat_hbm.at[pl.ds(q * E + g, CH)],
                             colb[r], sed[r])
            pltpu.async_copy(row_hbm.at[pl.ds(g, CH)], rowb[r], sed[r])
            pltpu.async_copy(e_hbm.at[pl.ds(g, CH)], eb[r], sed[r])

        def wait_edata(r):
            drain1i(sed[r], colb[r])
            drain1i(sed[r], rowb[r])
            drain1f(sed[r], eb[r])

        def gather(r):
            pltpu.async_copy(xcat_hbm.at[colb[r]], rowsb[r], sg[r])

        def scatter(r):
            pltpu.async_copy(rowsb[r], s1s.at[rowb[r]], ss[r], add=True)
            pltpu.async_copy(scab[r], s2s.at[rowb[r]], ss[r], add=True)

        def wait_scatter(r):
            drain2(ss[r], rowsb[r])
            drain2(ss[r], scab[r])

        edata(0, 0)
        wait_edata(0)
        gather(0)
        edata(1, 1)
        wait_edata(1)
        gather(1)
        drain2(sg[0], rowsb[0])
        scale(rowsb[0], scab[0], eb[0], CH)
        scatter(0)
        edata(2, 2)
        wait_edata(2)
        gather(2)
        drain2(sg[1], rowsb[1])
        scale(rowsb[1], scab[1], eb[1], CH)
        scatter(1)
        edata(3, 3)

        def steady(i, carry):
            for k in range(4):
                t = 2 + 4 * i + k
                r = (2 + k) % 4
                r1 = (3 + k) % 4
                r2 = k % 4
                wait_scatter(r2)
                edata(t + 2, r2)
                wait_edata(r1)
                gather(r1)
                drain2(sg[r], rowsb[r])
                scale(rowsb[r], scab[r], eb[r], CH)
                scatter(r)
            return carry

        lax.fori_loop(0, (NCH - 4) // 4, steady, 0)

        wait_edata(3)
        gather(3)
        drain2(sg[2], rowsb[2])
        scale(rowsb[2], scab[2], eb[2], CH)
        scatter(2)
        drain2(sg[3], rowsb[3])
        scale(rowsb[3], scab[3], eb[3], CH)
        scatter(3)
        for r in range(NR):
            wait_scatter(r)

        process_tail(q, base + NCH * CH)

        plsc.subcore_barrier()
        pltpu.sync_copy(s1s.at[pl.ds(r0, RPS)],
                        out1_hbm.at[pl.ds(q * NPAD + r0, RPS)])
        pltpu.sync_copy(s2s.at[pl.ds(r0, RPS)],
                        out2_hbm.at[pl.ds(q * NPAD + r0, RPS)])


_sc_edge_kernel = functools.partial(
    pl.kernel,
    out_type=(jax.ShapeDtypeStruct((2 * NP * NPAD, Q), jnp.float32),
              jax.ShapeDtypeStruct((2 * NP * NPAD, Q), jnp.float32)),
    mesh=plsc.VectorSubcoreMesh(core_axis_name="c", subcore_axis_name="s"),
    scratch_types=(
        [pltpu.VMEM((CH,), jnp.int32)] * 8
        + [pltpu.VMEM((CH,), jnp.float32)] * 4
        + [pltpu.VMEM((CH, Q), jnp.float32)] * 8
        + [
            pltpu.VMEM((TAIL,), jnp.int32),
            pltpu.VMEM((TAIL,), jnp.int32),
            pltpu.VMEM((TAIL,), jnp.float32),
            pltpu.VMEM((TAIL, Q), jnp.float32),
            pltpu.VMEM((TAIL, Q), jnp.float32),
            pltpu.VMEM((RPS, Q), jnp.float32),
            pltpu.VMEM_SHARED((NPAD, Q), jnp.float32),
            pltpu.VMEM_SHARED((NPAD, Q), jnp.float32),
        ]
        + [pltpu.SemaphoreType.DMA] * 13
    ),
    compiler_params=pltpu.CompilerParams(use_tc_tiling_on_sc=False,
                                         needs_layout_passes=False),
)(_sc_body)


JP = 4


def _tc_body(xT_ref, s1t_ref, s2t_ref, at_ref, bt_ref, wlt_ref, wtt_ref,
             out_ref, aggt_ref):
    t = pl.program_id(0)

    @pl.when(t == 0)
    def _init():
        aggt_ref[...] = (
            jnp.dot(at_ref[...], s2t_ref[...],
                    preferred_element_type=jnp.float32)
            + jnp.dot(bt_ref[...], s1t_ref[...],
                      preferred_element_type=jnp.float32))
        out_ref[...] = xT_ref[...] + jnp.dot(
            wlt_ref[...], xT_ref[...], preferred_element_type=jnp.float32)

    ypair = jnp.dot(wtt_ref[0], xT_ref[...],
                    preferred_element_type=jnp.float32)
    acc = out_ref[...]
    for u in range(JP):
        acc += (ypair[u * D:(u + 1) * D, :]
                * aggt_ref[pl.ds(t * JP + u, 1), :])
    out_ref[...] = acc


def _tc_update(xT, s1t, s2t, AT, BT, WlT, WtT):
    return pl.pallas_call(
        _tc_body,
        grid=(D // JP,),
        in_specs=[
            pl.BlockSpec((D, N), lambda t: (0, 0)),
            pl.BlockSpec((D, N), lambda t: (0, 0)),
            pl.BlockSpec((D, N), lambda t: (0, 0)),
            pl.BlockSpec((D, D), lambda t: (0, 0)),
            pl.BlockSpec((D, D), lambda t: (0, 0)),
            pl.BlockSpec((D, D), lambda t: (0, 0)),
            pl.BlockSpec((1, JP * D, D), lambda t: (t, 0, 0)),
        ],
        out_specs=pl.BlockSpec((D, N), lambda t: (0, 0)),
        out_shape=jax.ShapeDtypeStruct((D, N), jnp.float32),
        scratch_shapes=[pltpu.VMEM((D, N), jnp.float32)],
        compiler_params=pltpu.CompilerParams(
            dimension_semantics=("arbitrary",)),
    )(xT, s1t, s2t, AT, BT, WlT, WtT)


@jax.jit
def kernel(node_features, edge_index, edge_attr_tensor, node_attr_scalar_raw,
           W_tp_msg, W_lin_msg, W_tp_upd, W_lin_upd):
    x = node_features
    col = edge_index[1]
    row = edge_index[0]
    e = edge_attr_tensor[:, 3]

    colcat = jnp.concatenate(
        [col, col + N, col + 2 * N, col + 3 * N])
    xcat = jnp.concatenate(
        [x[:, 0:Q], x[:, Q:2 * Q], x[:, 2 * Q:3 * Q], x[:, 3 * Q:4 * Q]],
        axis=0)

    out1, out2 = _sc_edge_kernel(xcat, colcat, row, e)

    s1t = out1.reshape(4, NPAD, Q)[:, :N].transpose(0, 2, 1).reshape(D, N)
    s2t = out2.reshape(4, NPAD, Q)[:, :N].transpose(0, 2, 1).reshape(D, N)

    xT = x.T
    scale = 1.0 / (jnp.sqrt(float(D)) * float(D))
    AT = W_tp_msg[:, 0, :].T * scale
    BT = W_lin_msg.T * scale
    WlT = W_lin_upd.T / jnp.sqrt(float(D))
    WtT = W_tp_upd.transpose(1, 2, 0).reshape(D // JP, JP * D, D)

    outT = _tc_update(xT, s1t, s2t, AT, BT, WlT, WtT)
    return outT.T


# speedup vs baseline: 1.0307x; 1.0119x over previous
import functools
import jax
import jax.numpy as jnp
from jax import lax
from jax.experimental import pallas as pl
from jax.experimental.pallas import tpu as pltpu
from jax.experimental.pallas import tpu_sc as plsc

N = 10000
E = 320000
D = 128
Q = 32
NS = 16
NC = 2
NP = 2
EPS = E // NS
CH = 128
NCH = EPS // CH
TAIL = EPS - NCH * CH
NPAD = 10240
RPS = NPAD // NS


NR = 6


def _sc_body(xcat_hbm, colcat_hbm, row_hbm, e_hbm, out1_hbm, out2_hbm,
             col0, col1, col2, col3, col4, col5,
             row0, row1, row2, row3, row4, row5,
             e0, e1, e2, e3, e4, e5,
             rows0, rows1, rows2, rows3, rows4, rows5,
             sca0, sca1, sca2, sca3, sca4, sca5,
             col_t, row_t, e_t, rows_t, scaled_t,
             zbuf, s1s, s2s,
             sed0, sed1, sed2, sed3, sed4, sed5,
             sg0, sg1, sg2, sg3, sg4, sg5,
             ss0, ss1, ss2, ss3, ss4, ss5, sem):
    c = lax.axis_index("c")
    s = lax.axis_index("s")
    r0 = s * RPS
    base = s * EPS
    zv = jnp.zeros((16,), jnp.float32)

    colb = (col0, col1, col2, col3, col4, col5)
    rowb = (row0, row1, row2, row3, row4, row5)
    eb = (e0, e1, e2, e3, e4, e5)
    rowsb = (rows0, rows1, rows2, rows3, rows4, rows5)
    scab = (sca0, sca1, sca2, sca3, sca4, sca5)
    sed = (sed0, sed1, sed2, sed3, sed4, sed5)
    sg = (sg0, sg1, sg2, sg3, sg4, sg5)
    ss = (ss0, ss1, ss2, ss3, ss4, ss5)

    def zero_body(i, carry):
        for k in range(Q // 16):
            zbuf[i, pl.ds(16 * k, 16)] = zv
        return carry

    def drain1i(semr, ref):
        pltpu.make_async_copy(colcat_hbm.at[pl.ds(0, CH)], ref, semr).wait()

    def drain1f(semr, ref):
        pltpu.make_async_copy(e_hbm.at[pl.ds(0, CH)], ref, semr).wait()

    def drain2(semr, ref):
        pltpu.make_async_copy(xcat_hbm.at[pl.ds(0, CH)], ref, semr).wait()

    def scale(rowsref, scaref, eref, n):
        @plsc.parallel_loop(0, n // 16, 1, unroll=1)
        def scale_body(g):
            evec = eref[pl.ds(16 * g, 16)]
            for u in range(16):
                ev = lax.gather(
                    evec, jnp.full((16, 1), u, jnp.int32),
                    lax.GatherDimensionNumbers(
                        offset_dims=(), collapsed_slice_dims=(0,),
                        start_index_map=(0,)),
                    (1,), mode=lax.GatherScatterMode.PROMISE_IN_BOUNDS)
                i = 16 * g + u
                for k in range(Q // 16):
                    scaref[i, pl.ds(16 * k, 16)] = (
                        rowsref[i, pl.ds(16 * k, 16)] * ev)

    def process_tail(q, g):
        pltpu.sync_copy(colcat_hbm.at[pl.ds(q * E + g, TAIL)], col_t)
        pltpu.sync_copy(row_hbm.at[pl.ds(g, TAIL)], row_t)
        pltpu.sync_copy(e_hbm.at[pl.ds(g, TAIL)], e_t)
        pltpu.async_copy(xcat_hbm.at[col_t], rows_t, sem).wait()
        scale(rows_t, scaled_t, e_t, TAIL)
        pltpu.sync_copy(rows_t, s1s.at[row_t], add=True)
        pltpu.sync_copy(scaled_t, s2s.at[row_t], add=True)

    for p in range(NP):
        q = NP * p + c

        lax.fori_loop(0, RPS, zero_body, 0)
        pltpu.sync_copy(zbuf, s1s.at[pl.ds(r0, RPS)])
        pltpu.sync_copy(zbuf, s2s.at[pl.ds(r0, RPS)])
        plsc.subcore_barrier()

        def edata(t, r):
            g = base + t * CH
            pltpu.async_copy(colcat_hbm.at[pl.ds(q * E + g, CH)],
                             colb[r], sed[r])
            pltpu.async_copy(row_hbm.at[pl.ds(g, CH)], rowb[r], sed[r])
            pltpu.async_copy(e_hbm.at[pl.ds(g, CH)], eb[r], sed[r])

        def wait_edata(r):
            drain1i(sed[r], colb[r])
            drain1i(sed[r], rowb[r])
            drain1f(sed[r], eb[r])

        def gather(r):
            pltpu.async_copy(xcat_hbm.at[colb[r]], rowsb[r], sg[r])

        def scatter(r):
            pltpu.async_copy(rowsb[r], s1s.at[rowb[r]], ss[r], add=True)
            pltpu.async_copy(scab[r], s2s.at[rowb[r]], ss[r], add=True)

        def wait_scatter(r):
            drain2(ss[r], rowsb[r])
            drain2(ss[r], scab[r])

        edata(0, 0)
        wait_edata(0)
        gather(0)
        edata(1, 1)
        wait_edata(1)
        gather(1)
        edata(2, 2)
        for tp in range(3):
            wait_edata(tp + 2)
            gather(tp + 2)
            drain2(sg[tp], rowsb[tp])
            scale(rowsb[tp], scab[tp], eb[tp], CH)
            scatter(tp)
            edata(tp + 3, tp + 3)

        def steady(i, carry):
            for k in range(6):
                t = 3 + 6 * i + k
                r = (3 + k) % 6
                r2 = (5 + k) % 6
                r3 = k % 6
                wait_scatter(r3)
                edata(t + 3, r3)
                wait_edata(r2)
                gather(r2)
                drain2(sg[r], rowsb[r])
                scale(rowsb[r], scab[r], eb[r], CH)
                scatter(r)
            return carry

        lax.fori_loop(0, (NCH - 6) // 6, steady, 0)

        wait_edata(5)
        gather(5)
        for tp in range(3):
            r = 3 + tp
            drain2(sg[r], rowsb[r])
            scale(rowsb[r], scab[r], eb[r], CH)
            scatter(r)
        for r in range(NR):
            wait_scatter(r)

        process_tail(q, base + NCH * CH)

        plsc.subcore_barrier()
        pltpu.sync_copy(s1s.at[pl.ds(r0, RPS)],
                        out1_hbm.at[pl.ds(q * NPAD + r0, RPS)])
        pltpu.sync_copy(s2s.at[pl.ds(r0, RPS)],
                        out2_hbm.at[pl.ds(q * NPAD + r0, RPS)])


_sc_edge_kernel = functools.partial(
    pl.kernel,
    out_type=(jax.ShapeDtypeStruct((2 * NP * NPAD, Q), jnp.float32),
              jax.ShapeDtypeStruct((2 * NP * NPAD, Q), jnp.float32)),
    mesh=plsc.VectorSubcoreMesh(core_axis_name="c", subcore_axis_name="s"),
    scratch_types=(
        [pltpu.VMEM((CH,), jnp.int32)] * 12
        + [pltpu.VMEM((CH,), jnp.float32)] * 6
        + [pltpu.VMEM((CH, Q), jnp.float32)] * 12
        + [
            pltpu.VMEM((TAIL,), jnp.int32),
            pltpu.VMEM((TAIL,), jnp.int32),
            pltpu.VMEM((TAIL,), jnp.float32),
            pltpu.VMEM((TAIL, Q), jnp.float32),
            pltpu.VMEM((TAIL, Q), jnp.float32),
            pltpu.VMEM((RPS, Q), jnp.float32),
            pltpu.VMEM_SHARED((NPAD, Q), jnp.float32),
            pltpu.VMEM_SHARED((NPAD, Q), jnp.float32),
        ]
        + [pltpu.SemaphoreType.DMA] * 19
    ),
    compiler_params=pltpu.CompilerParams(use_tc_tiling_on_sc=False,
                                         needs_layout_passes=False),
)(_sc_body)


JP = 4


def _tc_body(xT_ref, s1t_ref, s2t_ref, at_ref, bt_ref, wlt_ref, wtt_ref,
             out_ref, aggt_ref):
    t = pl.program_id(0)

    @pl.when(t == 0)
    def _init():
        aggt_ref[...] = (
            jnp.dot(at_ref[...], s2t_ref[...],
                    preferred_element_type=jnp.float32)
            + jnp.dot(bt_ref[...], s1t_ref[...],
                      preferred_element_type=jnp.float32))
        out_ref[...] = xT_ref[...] + jnp.dot(
            wlt_ref[...], xT_ref[...], preferred_element_type=jnp.float32)

    ypair = jnp.dot(wtt_ref[0], xT_ref[...],
                    preferred_element_type=jnp.float32)
    acc = out_ref[...]
    for u in range(JP):
        acc += (ypair[u * D:(u + 1) * D, :]
                * aggt_ref[pl.ds(t * JP + u, 1), :])
    out_ref[...] = acc


def _tc_update(xT, s1t, s2t, AT, BT, WlT, WtT):
    return pl.pallas_call(
        _tc_body,
        grid=(D // JP,),
        in_specs=[
            pl.BlockSpec((D, N), lambda t: (0, 0)),
            pl.BlockSpec((D, N), lambda t: (0, 0)),
            pl.BlockSpec((D, N), lambda t: (0, 0)),
            pl.BlockSpec((D, D), lambda t: (0, 0)),
            pl.BlockSpec((D, D), lambda t: (0, 0)),
            pl.BlockSpec((D, D), lambda t: (0, 0)),
            pl.BlockSpec((1, JP * D, D), lambda t: (t, 0, 0)),
        ],
        out_specs=pl.BlockSpec((D, N), lambda t: (0, 0)),
        out_shape=jax.ShapeDtypeStruct((D, N), jnp.float32),
        scratch_shapes=[pltpu.VMEM((D, N), jnp.float32)],
        compiler_params=pltpu.CompilerParams(
            dimension_semantics=("arbitrary",)),
    )(xT, s1t, s2t, AT, BT, WlT, WtT)


@jax.jit
def kernel(node_features, edge_index, edge_attr_tensor, node_attr_scalar_raw,
           W_tp_msg, W_lin_msg, W_tp_upd, W_lin_upd):
    x = node_features
    col = edge_index[1]
    row = edge_index[0]
    e = edge_attr_tensor[:, 3]

    colcat = jnp.concatenate(
        [col, col + N, col + 2 * N, col + 3 * N])
    xcat = jnp.concatenate(
        [x[:, 0:Q], x[:, Q:2 * Q], x[:, 2 * Q:3 * Q], x[:, 3 * Q:4 * Q]],
        axis=0)

    out1, out2 = _sc_edge_kernel(xcat, colcat, row, e)

    s1t = out1.reshape(4, NPAD, Q)[:, :N].transpose(0, 2, 1).reshape(D, N)
    s2t = out2.reshape(4, NPAD, Q)[:, :N].transpose(0, 2, 1).reshape(D, N)

    xT = x.T
    scale = 1.0 / (jnp.sqrt(float(D)) * float(D))
    AT = W_tp_msg[:, 0, :].T * scale
    BT = W_lin_msg.T * scale
    WlT = W_lin_upd.T / jnp.sqrt(float(D))
    WtT = W_tp_upd.transpose(1, 2, 0).reshape(D // JP, JP * D, D)

    outT = _tc_update(xT, s1t, s2t, AT, BT, WlT, WtT)
    return outT.T
